# baseline (device time: 778544 ns/iter reference)
import jax
import jax.numpy as jnp
from jax import lax
from jax.experimental import pallas as pl
from jax.experimental.pallas import tpu as pltpu

N_DEV = 16
K_SLOTS = 4


def kernel(x, w_mat):
    m, k_shard = x.shape
    _, n_out = w_mat.shape
    ch = m // N_DEV

    def body(x_ref, w_ref, out_ref, comm_ref,
             rs_send_sems, rs_recv_sems, credit_sems,
             ag_send_sems, ag_recv_sems):
        my = lax.axis_index("i")
        left = (my - 1) % N_DEV
        right = (my + 1) % N_DEV

        barrier_sem = pltpu.get_barrier_semaphore()
        for nbr in (left, right):
            pl.semaphore_signal(
                barrier_sem, inc=1,
                device_id=(nbr,), device_id_type=pl.DeviceIdType.MESH,
            )
        pl.semaphore_wait(barrier_sem, 2)

        for c in range(N_DEV):
            out_ref[pl.ds(c * ch, ch), :] = jnp.dot(
                x_ref[pl.ds(c * ch, ch), :], w_ref[...],
                preferred_element_type=jnp.float32,
            )

        for s in range(N_DEV - 1):
            slot = s % K_SLOTS
            send_c = (my - s) % N_DEV
            recv_c = (my - s - 1) % N_DEV
            if s >= K_SLOTS:
                pl.semaphore_wait(credit_sems.at[slot], 1)
            rdma = pltpu.make_async_remote_copy(
                src_ref=out_ref.at[pl.ds(send_c * ch, ch), :],
                dst_ref=comm_ref.at[slot],
                send_sem=rs_send_sems.at[slot],
                recv_sem=rs_recv_sems.at[slot],
                device_id=(right,),
                device_id_type=pl.DeviceIdType.MESH,
            )
            rdma.start()
            rdma.wait()
            out_ref[pl.ds(recv_c * ch, ch), :] = (
                out_ref[pl.ds(recv_c * ch, ch), :] + comm_ref[slot]
            )
            if s + K_SLOTS <= N_DEV - 2:
                pl.semaphore_signal(
                    credit_sems.at[slot], inc=1,
                    device_id=(left,), device_id_type=pl.DeviceIdType.MESH,
                )

        own = (my + 1) % N_DEV
        z = out_ref[pl.ds(own * ch, ch), :]
        out_ref[pl.ds(own * ch, ch), :] = z / (
            1.0 + jnp.exp(-jnp.clip(z, -60.0, 60.0))
        )

        for s in range(N_DEV - 1):
            send_c = (my + 1 - s) % N_DEV
            rdma = pltpu.make_async_remote_copy(
                src_ref=out_ref.at[pl.ds(send_c * ch, ch), :],
                dst_ref=out_ref.at[pl.ds(send_c * ch, ch), :],
                send_sem=ag_send_sems.at[s],
                recv_sem=ag_recv_sems.at[s],
                device_id=(right,),
                device_id_type=pl.DeviceIdType.MESH,
            )
            rdma.start()
            rdma.wait()

    return pl.pallas_call(
        body,
        out_shape=jax.ShapeDtypeStruct((m, n_out), jnp.float32),
        in_specs=[
            pl.BlockSpec(memory_space=pltpu.VMEM),
            pl.BlockSpec(memory_space=pltpu.VMEM),
        ],
        out_specs=pl.BlockSpec(memory_space=pltpu.VMEM),
        scratch_shapes=[
            pltpu.VMEM((K_SLOTS, ch, n_out), jnp.float32),
            pltpu.SemaphoreType.DMA((K_SLOTS,)),
            pltpu.SemaphoreType.DMA((K_SLOTS,)),
            pltpu.SemaphoreType.REGULAR((K_SLOTS,)),
            pltpu.SemaphoreType.DMA((N_DEV - 1,)),
            pltpu.SemaphoreType.DMA((N_DEV - 1,)),
        ],
        compiler_params=pltpu.CompilerParams(
            collective_id=0,
            vmem_limit_bytes=60 * 1024 * 1024,
        ),
    )(x, w_mat)


# device time: 481874 ns/iter; 1.6157x vs baseline; 1.6157x over previous
import jax
import jax.numpy as jnp
from jax import lax
from jax.experimental import pallas as pl
from jax.experimental.pallas import tpu as pltpu

N_DEV = 16
K_SLOTS = 4


def kernel(x, w_mat):
    m, k_shard = x.shape
    _, n_out = w_mat.shape
    ch = m // N_DEV
    nh = n_out // 2

    def body(x_ref, w_ref, out_ref, cw_ref, ccw_ref,
             cw_send_sems, cw_recv_sems, cw_credit_sems,
             ccw_send_sems, ccw_recv_sems, ccw_credit_sems,
             ag_cw_send_sems, ag_cw_recv_sems,
             ag_ccw_send_sems, ag_ccw_recv_sems):
        my = lax.axis_index("i")
        left = (my - 1) % N_DEV
        right = (my + 1) % N_DEV

        barrier_sem = pltpu.get_barrier_semaphore()
        for nbr in (left, right):
            pl.semaphore_signal(
                barrier_sem, inc=1,
                device_id=(nbr,), device_id_type=pl.DeviceIdType.MESH,
            )
        pl.semaphore_wait(barrier_sem, 2)

        for c in range(N_DEV):
            out_ref[pl.ds(c * ch, ch), :] = jnp.dot(
                x_ref[pl.ds(c * ch, ch), :], w_ref[...],
                preferred_element_type=jnp.float32,
            )

        lo = pl.ds(0, nh)
        hi = pl.ds(nh, nh)

        for s in range(N_DEV - 1):
            slot = s % K_SLOTS
            cw_send_c = (my - s) % N_DEV
            cw_recv_c = (my - s - 1) % N_DEV
            ccw_send_c = (my + s) % N_DEV
            ccw_recv_c = (my + s + 1) % N_DEV
            if s >= K_SLOTS:
                pl.semaphore_wait(cw_credit_sems.at[slot], 1)
                pl.semaphore_wait(ccw_credit_sems.at[slot], 1)
            cw = pltpu.make_async_remote_copy(
                src_ref=out_ref.at[pl.ds(cw_send_c * ch, ch), lo],
                dst_ref=cw_ref.at[slot],
                send_sem=cw_send_sems.at[slot],
                recv_sem=cw_recv_sems.at[slot],
                device_id=(right,),
                device_id_type=pl.DeviceIdType.MESH,
            )
            ccw = pltpu.make_async_remote_copy(
                src_ref=out_ref.at[pl.ds(ccw_send_c * ch, ch), hi],
                dst_ref=ccw_ref.at[slot],
                send_sem=ccw_send_sems.at[slot],
                recv_sem=ccw_recv_sems.at[slot],
                device_id=(left,),
                device_id_type=pl.DeviceIdType.MESH,
            )
            cw.start()
            ccw.start()
            cw.wait()
            ccw.wait()
            out_ref[pl.ds(cw_recv_c * ch, ch), lo] = (
                out_ref[pl.ds(cw_recv_c * ch, ch), lo] + cw_ref[slot]
            )
            out_ref[pl.ds(ccw_recv_c * ch, ch), hi] = (
                out_ref[pl.ds(ccw_recv_c * ch, ch), hi] + ccw_ref[slot]
            )
            if s + K_SLOTS <= N_DEV - 2:
                pl.semaphore_signal(
                    cw_credit_sems.at[slot], inc=1,
                    device_id=(left,), device_id_type=pl.DeviceIdType.MESH,
                )
                pl.semaphore_signal(
                    ccw_credit_sems.at[slot], inc=1,
                    device_id=(right,), device_id_type=pl.DeviceIdType.MESH,
                )

        own_cw = (my + 1) % N_DEV
        own_ccw = (my - 1) % N_DEV
        z = out_ref[pl.ds(own_cw * ch, ch), lo]
        out_ref[pl.ds(own_cw * ch, ch), lo] = z / (
            1.0 + jnp.exp(-jnp.clip(z, -60.0, 60.0))
        )
        z = out_ref[pl.ds(own_ccw * ch, ch), hi]
        out_ref[pl.ds(own_ccw * ch, ch), hi] = z / (
            1.0 + jnp.exp(-jnp.clip(z, -60.0, 60.0))
        )

        for s in range(N_DEV - 1):
            cw_send_c = (my + 1 - s) % N_DEV
            ccw_send_c = (my - 1 + s) % N_DEV
            cw = pltpu.make_async_remote_copy(
                src_ref=out_ref.at[pl.ds(cw_send_c * ch, ch), lo],
                dst_ref=out_ref.at[pl.ds(cw_send_c * ch, ch), lo],
                send_sem=ag_cw_send_sems.at[s],
                recv_sem=ag_cw_recv_sems.at[s],
                device_id=(right,),
                device_id_type=pl.DeviceIdType.MESH,
            )
            ccw = pltpu.make_async_remote_copy(
                src_ref=out_ref.at[pl.ds(ccw_send_c * ch, ch), hi],
                dst_ref=out_ref.at[pl.ds(ccw_send_c * ch, ch), hi],
                send_sem=ag_ccw_send_sems.at[s],
                recv_sem=ag_ccw_recv_sems.at[s],
                device_id=(left,),
                device_id_type=pl.DeviceIdType.MESH,
            )
            cw.start()
            ccw.start()
            cw.wait()
            ccw.wait()

    return pl.pallas_call(
        body,
        out_shape=jax.ShapeDtypeStruct((m, n_out), jnp.float32),
        in_specs=[
            pl.BlockSpec(memory_space=pltpu.VMEM),
            pl.BlockSpec(memory_space=pltpu.VMEM),
        ],
        out_specs=pl.BlockSpec(memory_space=pltpu.VMEM),
        scratch_shapes=[
            pltpu.VMEM((K_SLOTS, ch, nh), jnp.float32),
            pltpu.VMEM((K_SLOTS, ch, nh), jnp.float32),
            pltpu.SemaphoreType.DMA((K_SLOTS,)),
            pltpu.SemaphoreType.DMA((K_SLOTS,)),
            pltpu.SemaphoreType.REGULAR((K_SLOTS,)),
            pltpu.SemaphoreType.DMA((K_SLOTS,)),
            pltpu.SemaphoreType.DMA((K_SLOTS,)),
            pltpu.SemaphoreType.REGULAR((K_SLOTS,)),
            pltpu.SemaphoreType.DMA((N_DEV - 1,)),
            pltpu.SemaphoreType.DMA((N_DEV - 1,)),
            pltpu.SemaphoreType.DMA((N_DEV - 1,)),
            pltpu.SemaphoreType.DMA((N_DEV - 1,)),
        ],
        compiler_params=pltpu.CompilerParams(
            collective_id=0,
            vmem_limit_bytes=60 * 1024 * 1024,
        ),
    )(x, w_mat)


# device time: 389762 ns/iter; 1.9975x vs baseline; 1.2363x over previous
import jax
import jax.numpy as jnp
from jax import lax
from jax.experimental import pallas as pl
from jax.experimental.pallas import tpu as pltpu

N_DEV = 16
SUB = 2
K_SLOTS = 4
NSLOT = K_SLOTS * SUB
NSTEP = N_DEV - 1
NSUB = NSTEP * SUB


def kernel(x, w_mat):
    m, k_shard = x.shape
    _, n_out = w_mat.shape
    ch = m // N_DEV
    sch = ch // SUB
    nh = n_out // 2

    def body(x_ref, w_ref, out_ref, cw_ref, ccw_ref,
             cw_send_sems, cw_recv_sems, cw_credit_sems,
             ccw_send_sems, ccw_recv_sems, ccw_credit_sems,
             ag_cw_send_sems, ag_cw_recv_sems,
             ag_ccw_send_sems, ag_ccw_recv_sems):
        my = lax.axis_index("i")
        left = (my - 1) % N_DEV
        right = (my + 1) % N_DEV

        lo = pl.ds(0, nh)
        hi = pl.ds(nh, nh)

        def rows(c, sub):
            return pl.ds(c * ch + sub * sch, sch)

        def rs_rdma(direction, t, chunk, sub):
            slot = t % NSLOT
            if direction == "cw":
                return pltpu.make_async_remote_copy(
                    src_ref=out_ref.at[rows(chunk, sub), lo],
                    dst_ref=cw_ref.at[slot],
                    send_sem=cw_send_sems.at[t],
                    recv_sem=cw_recv_sems.at[t],
                    device_id=(right,),
                    device_id_type=pl.DeviceIdType.MESH,
                )
            return pltpu.make_async_remote_copy(
                src_ref=out_ref.at[rows(chunk, sub), hi],
                dst_ref=ccw_ref.at[slot],
                send_sem=ccw_send_sems.at[t],
                recv_sem=ccw_recv_sems.at[t],
                device_id=(left,),
                device_id_type=pl.DeviceIdType.MESH,
            )

        def rs_send(direction, s, sub):
            t = s * SUB + sub
            if t >= NSLOT:
                sems = cw_credit_sems if direction == "cw" else ccw_credit_sems
                pl.semaphore_wait(sems.at[t % NSLOT], 1)
            c = (my - s) % N_DEV if direction == "cw" else (my + s) % N_DEV
            rs_rdma(direction, t, c, sub).start()

        barrier_sem = pltpu.get_barrier_semaphore()
        for nbr in (left, right):
            pl.semaphore_signal(
                barrier_sem, inc=1,
                device_id=(nbr,), device_id_type=pl.DeviceIdType.MESH,
            )
        pl.semaphore_wait(barrier_sem, 2)

        def gemm_chunk(c):
            out_ref[pl.ds(c * ch, ch), :] = jnp.dot(
                x_ref[pl.ds(c * ch, ch), :], w_ref[...],
                preferred_element_type=jnp.float32,
            )

        gemm_chunk(my)
        for sub in range(SUB):
            rs_send("cw", 0, sub)
            rs_send("ccw", 0, sub)
        offs = [o for k in range(1, N_DEV // 2) for o in (-k, k)] + [N_DEV // 2]
        for o in offs:
            gemm_chunk((my + o) % N_DEV)

        for s in range(NSTEP):
            for sub in range(SUB):
                t = s * SUB + sub
                cw_recv_c = (my - s - 1) % N_DEV
                ccw_recv_c = (my + s + 1) % N_DEV
                rs_rdma("cw", t, cw_recv_c, sub).wait_recv()
                rs_rdma("ccw", t, ccw_recv_c, sub).wait_recv()
                out_ref[rows(cw_recv_c, sub), lo] = (
                    out_ref[rows(cw_recv_c, sub), lo] + cw_ref[t % NSLOT]
                )
                out_ref[rows(ccw_recv_c, sub), hi] = (
                    out_ref[rows(ccw_recv_c, sub), hi] + ccw_ref[t % NSLOT]
                )
                if t + NSLOT <= NSUB - 1:
                    pl.semaphore_signal(
                        cw_credit_sems.at[t % NSLOT], inc=1,
                        device_id=(left,), device_id_type=pl.DeviceIdType.MESH,
                    )
                    pl.semaphore_signal(
                        ccw_credit_sems.at[t % NSLOT], inc=1,
                        device_id=(right,), device_id_type=pl.DeviceIdType.MESH,
                    )
                if s + 1 < NSTEP:
                    rs_send("cw", s + 1, sub)
                    rs_send("ccw", s + 1, sub)

        for s in range(NSTEP):
            for sub in range(SUB):
                t = s * SUB + sub
                c_cw = (my - s) % N_DEV
                c_ccw = (my + s) % N_DEV
                rs_rdma("cw", t, c_cw, sub).wait_send()
                rs_rdma("ccw", t, c_ccw, sub).wait_send()

        own_cw = (my + 1) % N_DEV
        own_ccw = (my - 1) % N_DEV
        z = out_ref[pl.ds(own_cw * ch, ch), lo]
        out_ref[pl.ds(own_cw * ch, ch), lo] = z / (
            1.0 + jnp.exp(-jnp.clip(z, -60.0, 60.0))
        )
        z = out_ref[pl.ds(own_ccw * ch, ch), hi]
        out_ref[pl.ds(own_ccw * ch, ch), hi] = z / (
            1.0 + jnp.exp(-jnp.clip(z, -60.0, 60.0))
        )

        def ag_rdma(direction, s, sub):
            t = s * SUB + sub
            if direction == "cw":
                c = (my + 1 - s) % N_DEV
                return pltpu.make_async_remote_copy(
                    src_ref=out_ref.at[rows(c, sub), lo],
                    dst_ref=out_ref.at[rows(c, sub), lo],
                    send_sem=ag_cw_send_sems.at[t],
                    recv_sem=ag_cw_recv_sems.at[t],
                    device_id=(right,),
                    device_id_type=pl.DeviceIdType.MESH,
                )
            c = (my - 1 + s) % N_DEV
            return pltpu.make_async_remote_copy(
                src_ref=out_ref.at[rows(c, sub), hi],
                dst_ref=out_ref.at[rows(c, sub), hi],
                send_sem=ag_ccw_send_sems.at[t],
                recv_sem=ag_ccw_recv_sems.at[t],
                device_id=(left,),
                device_id_type=pl.DeviceIdType.MESH,
            )

        for sub in range(SUB):
            ag_rdma("cw", 0, sub).start()
            ag_rdma("ccw", 0, sub).start()
        for s in range(NSTEP):
            for sub in range(SUB):
                ag_rdma("cw", s, sub).wait_recv()
                ag_rdma("ccw", s, sub).wait_recv()
                if s + 1 < NSTEP:
                    ag_rdma("cw", s + 1, sub).start()
                    ag_rdma("ccw", s + 1, sub).start()

        for s in range(NSTEP):
            for sub in range(SUB):
                ag_rdma("cw", s, sub).wait_send()
                ag_rdma("ccw", s, sub).wait_send()

    return pl.pallas_call(
        body,
        out_shape=jax.ShapeDtypeStruct((m, n_out), jnp.float32),
        in_specs=[
            pl.BlockSpec(memory_space=pltpu.VMEM),
            pl.BlockSpec(memory_space=pltpu.VMEM),
        ],
        out_specs=pl.BlockSpec(memory_space=pltpu.VMEM),
        scratch_shapes=[
            pltpu.VMEM((NSLOT, sch, nh), jnp.float32),
            pltpu.VMEM((NSLOT, sch, nh), jnp.float32),
            pltpu.SemaphoreType.DMA((NSUB,)),
            pltpu.SemaphoreType.DMA((NSUB,)),
            pltpu.SemaphoreType.REGULAR((NSLOT,)),
            pltpu.SemaphoreType.DMA((NSUB,)),
            pltpu.SemaphoreType.DMA((NSUB,)),
            pltpu.SemaphoreType.REGULAR((NSLOT,)),
            pltpu.SemaphoreType.DMA((NSUB,)),
            pltpu.SemaphoreType.DMA((NSUB,)),
            pltpu.SemaphoreType.DMA((NSUB,)),
            pltpu.SemaphoreType.DMA((NSUB,)),
        ],
        compiler_params=pltpu.CompilerParams(
            collective_id=0,
            vmem_limit_bytes=60 * 1024 * 1024,
        ),
    )(x, w_mat)


# device time: 387250 ns/iter; 2.0104x vs baseline; 1.0065x over previous
import jax
import jax.numpy as jnp
from jax import lax
from jax.experimental import pallas as pl
from jax.experimental.pallas import tpu as pltpu

N_DEV = 16
SUB = 4
K_SLOTS = 4
NSLOT = K_SLOTS * SUB
NSTEP = N_DEV - 1
NSUB = NSTEP * SUB


def kernel(x, w_mat):
    m, k_shard = x.shape
    _, n_out = w_mat.shape
    ch = m // N_DEV
    sch = ch // SUB
    nh = n_out // 2

    def body(x_ref, w_ref, out_ref, cw_ref, ccw_ref,
             cw_send_sems, cw_recv_sems, cw_credit_sems,
             ccw_send_sems, ccw_recv_sems, ccw_credit_sems,
             ag_cw_send_sems, ag_cw_recv_sems,
             ag_ccw_send_sems, ag_ccw_recv_sems):
        my = lax.axis_index("i")
        left = (my - 1) % N_DEV
        right = (my + 1) % N_DEV

        lo = pl.ds(0, nh)
        hi = pl.ds(nh, nh)

        def rows(c, sub):
            return pl.ds(c * ch + sub * sch, sch)

        def silu(v):
            return v / (1.0 + jnp.exp(-jnp.clip(v, -60.0, 60.0)))

        def rs_rdma(direction, t, chunk, sub):
            slot = t % NSLOT
            if direction == "cw":
                return pltpu.make_async_remote_copy(
                    src_ref=out_ref.at[rows(chunk, sub), lo],
                    dst_ref=cw_ref.at[slot],
                    send_sem=cw_send_sems.at[t],
                    recv_sem=cw_recv_sems.at[t],
                    device_id=(right,),
                    device_id_type=pl.DeviceIdType.MESH,
                )
            return pltpu.make_async_remote_copy(
                src_ref=out_ref.at[rows(chunk, sub), hi],
                dst_ref=ccw_ref.at[slot],
                send_sem=ccw_send_sems.at[t],
                recv_sem=ccw_recv_sems.at[t],
                device_id=(left,),
                device_id_type=pl.DeviceIdType.MESH,
            )

        def rs_send(direction, s, sub):
            t = s * SUB + sub
            if t >= NSLOT:
                sems = cw_credit_sems if direction == "cw" else ccw_credit_sems
                pl.semaphore_wait(sems.at[t % NSLOT], 1)
            c = (my - s) % N_DEV if direction == "cw" else (my + s) % N_DEV
            rs_rdma(direction, t, c, sub).start()

        def ag_rdma(direction, s, sub):
            t = s * SUB + sub
            if direction == "cw":
                c = (my + 1 - s) % N_DEV
                return pltpu.make_async_remote_copy(
                    src_ref=out_ref.at[rows(c, sub), lo],
                    dst_ref=out_ref.at[rows(c, sub), lo],
                    send_sem=ag_cw_send_sems.at[t],
                    recv_sem=ag_cw_recv_sems.at[t],
                    device_id=(right,),
                    device_id_type=pl.DeviceIdType.MESH,
                )
            c = (my - 1 + s) % N_DEV
            return pltpu.make_async_remote_copy(
                src_ref=out_ref.at[rows(c, sub), hi],
                dst_ref=out_ref.at[rows(c, sub), hi],
                send_sem=ag_ccw_send_sems.at[t],
                recv_sem=ag_ccw_recv_sems.at[t],
                device_id=(left,),
                device_id_type=pl.DeviceIdType.MESH,
            )

        barrier_sem = pltpu.get_barrier_semaphore()
        for nbr in (left, right):
            pl.semaphore_signal(
                barrier_sem, inc=1,
                device_id=(nbr,), device_id_type=pl.DeviceIdType.MESH,
            )
        pl.semaphore_wait(barrier_sem, 2)

        def gemm_chunk(c):
            out_ref[pl.ds(c * ch, ch), :] = jnp.dot(
                x_ref[pl.ds(c * ch, ch), :], w_ref[...],
                preferred_element_type=jnp.float32,
            )

        gemm_chunk(my)
        for sub in range(SUB):
            rs_send("cw", 0, sub)
            rs_send("ccw", 0, sub)
        offs = [o for k in range(1, N_DEV // 2) for o in (-k, k)] + [N_DEV // 2]
        for o in offs:
            gemm_chunk((my + o) % N_DEV)

        for s in range(NSTEP):
            last = s == NSTEP - 1
            for sub in range(SUB):
                t = s * SUB + sub
                cw_recv_c = (my - s - 1) % N_DEV
                ccw_recv_c = (my + s + 1) % N_DEV
                rs_rdma("cw", t, cw_recv_c, sub).wait_recv()
                acc = out_ref[rows(cw_recv_c, sub), lo] + cw_ref[t % NSLOT]
                out_ref[rows(cw_recv_c, sub), lo] = silu(acc) if last else acc
                if not last:
                    rs_send("cw", s + 1, sub)
                else:
                    ag_rdma("cw", 0, sub).start()
                rs_rdma("ccw", t, ccw_recv_c, sub).wait_recv()
                acc = out_ref[rows(ccw_recv_c, sub), hi] + ccw_ref[t % NSLOT]
                out_ref[rows(ccw_recv_c, sub), hi] = silu(acc) if last else acc
                if not last:
                    rs_send("ccw", s + 1, sub)
                else:
                    ag_rdma("ccw", 0, sub).start()
                if t + NSLOT <= NSUB - 1:
                    pl.semaphore_signal(
                        cw_credit_sems.at[t % NSLOT], inc=1,
                        device_id=(left,), device_id_type=pl.DeviceIdType.MESH,
                    )
                    pl.semaphore_signal(
                        ccw_credit_sems.at[t % NSLOT], inc=1,
                        device_id=(right,), device_id_type=pl.DeviceIdType.MESH,
                    )

        for s in range(NSTEP):
            for sub in range(SUB):
                ag_rdma("cw", s, sub).wait_recv()
                if s + 1 < NSTEP:
                    ag_rdma("cw", s + 1, sub).start()
                ag_rdma("ccw", s, sub).wait_recv()
                if s + 1 < NSTEP:
                    ag_rdma("ccw", s + 1, sub).start()

        for s in range(NSTEP):
            for sub in range(SUB):
                t = s * SUB + sub
                rs_rdma("cw", t, (my - s) % N_DEV, sub).wait_send()
                rs_rdma("ccw", t, (my + s) % N_DEV, sub).wait_send()
                ag_rdma("cw", s, sub).wait_send()
                ag_rdma("ccw", s, sub).wait_send()

    return pl.pallas_call(
        body,
        out_shape=jax.ShapeDtypeStruct((m, n_out), jnp.float32),
        in_specs=[
            pl.BlockSpec(memory_space=pltpu.VMEM),
            pl.BlockSpec(memory_space=pltpu.VMEM),
        ],
        out_specs=pl.BlockSpec(memory_space=pltpu.VMEM),
        scratch_shapes=[
            pltpu.VMEM((NSLOT, sch, nh), jnp.float32),
            pltpu.VMEM((NSLOT, sch, nh), jnp.float32),
            pltpu.SemaphoreType.DMA((NSUB,)),
            pltpu.SemaphoreType.DMA((NSUB,)),
            pltpu.SemaphoreType.REGULAR((NSLOT,)),
            pltpu.SemaphoreType.DMA((NSUB,)),
            pltpu.SemaphoreType.DMA((NSUB,)),
            pltpu.SemaphoreType.REGULAR((NSLOT,)),
            pltpu.SemaphoreType.DMA((NSUB,)),
            pltpu.SemaphoreType.DMA((NSUB,)),
            pltpu.SemaphoreType.DMA((NSUB,)),
            pltpu.SemaphoreType.DMA((NSUB,)),
        ],
        compiler_params=pltpu.CompilerParams(
            collective_id=0,
            vmem_limit_bytes=60 * 1024 * 1024,
        ),
    )(x, w_mat)


# device time: 304034 ns/iter; 2.5607x vs baseline; 1.2737x over previous
import jax
import jax.numpy as jnp
from jax import lax
from jax.experimental import pallas as pl
from jax.experimental.pallas import tpu as pltpu

N_DEV = 16
SUB = 4
K_SLOTS = 4
NSLOT = K_SLOTS * SUB
NSTEP = N_DEV - 1
NSUB = NSTEP * SUB


def kernel(x, w_mat):
    m, k_shard = x.shape
    _, n_out = w_mat.shape
    ch = m // N_DEV
    sch = ch // SUB
    nh = n_out // 2

    def body(x_ref, w_ref, out_ref, cw_ref, ccw_ref,
             ag_cw_ref, ag_ccw_ref, seed_cw_ref, seed_ccw_ref,
             cw_send_sems, cw_recv_sems, cw_credit_sems,
             ccw_send_sems, ccw_recv_sems, ccw_credit_sems,
             ag_cw_send_sems, ag_cw_recv_sems, ag_cw_credit_sems,
             ag_ccw_send_sems, ag_ccw_recv_sems, ag_ccw_credit_sems):
        my = lax.axis_index("i")
        left = (my - 1) % N_DEV
        right = (my + 1) % N_DEV

        lo = pl.ds(0, nh)
        hi = pl.ds(nh, nh)

        def rows(c, sub):
            return pl.ds(c * ch + sub * sch, sch)

        def silu(v):
            return v / (1.0 + jnp.exp(-jnp.clip(v, -60.0, 60.0)))

        def rs_rdma(direction, t, chunk, sub):
            slot = t % NSLOT
            if direction == "cw":
                return pltpu.make_async_remote_copy(
                    src_ref=out_ref.at[rows(chunk, sub), lo],
                    dst_ref=cw_ref.at[slot],
                    send_sem=cw_send_sems.at[t],
                    recv_sem=cw_recv_sems.at[t],
                    device_id=(right,),
                    device_id_type=pl.DeviceIdType.MESH,
                )
            return pltpu.make_async_remote_copy(
                src_ref=out_ref.at[rows(chunk, sub), hi],
                dst_ref=ccw_ref.at[slot],
                send_sem=ccw_send_sems.at[t],
                recv_sem=ccw_recv_sems.at[t],
                device_id=(left,),
                device_id_type=pl.DeviceIdType.MESH,
            )

        def rs_send(direction, s, sub):
            t = s * SUB + sub
            if t >= NSLOT:
                sems = cw_credit_sems if direction == "cw" else ccw_credit_sems
                pl.semaphore_wait(sems.at[t % NSLOT], 1)
            c = (my - s) % N_DEV if direction == "cw" else (my + s) % N_DEV
            rs_rdma(direction, t, c, sub).start()

        def ag_rdma(direction, s, sub):
            t = s * SUB + sub
            if direction == "cw":
                src = (seed_cw_ref.at[sub] if s == 0
                       else ag_cw_ref.at[(t - SUB) % NSLOT])
                return pltpu.make_async_remote_copy(
                    src_ref=src,
                    dst_ref=ag_cw_ref.at[t % NSLOT],
                    send_sem=ag_cw_send_sems.at[t],
                    recv_sem=ag_cw_recv_sems.at[t],
                    device_id=(right,),
                    device_id_type=pl.DeviceIdType.MESH,
                )
            src = (seed_ccw_ref.at[sub] if s == 0
                   else ag_ccw_ref.at[(t - SUB) % NSLOT])
            return pltpu.make_async_remote_copy(
                src_ref=src,
                dst_ref=ag_ccw_ref.at[t % NSLOT],
                send_sem=ag_ccw_send_sems.at[t],
                recv_sem=ag_ccw_recv_sems.at[t],
                device_id=(left,),
                device_id_type=pl.DeviceIdType.MESH,
            )

        def ag_send(direction, s, sub):
            t = s * SUB + sub
            if t >= NSLOT:
                sems = (ag_cw_credit_sems if direction == "cw"
                        else ag_ccw_credit_sems)
                pl.semaphore_wait(sems.at[t % NSLOT], 1)
            ag_rdma(direction, s, sub).start()

        barrier_sem = pltpu.get_barrier_semaphore()
        for nbr in (left, right):
            pl.semaphore_signal(
                barrier_sem, inc=1,
                device_id=(nbr,), device_id_type=pl.DeviceIdType.MESH,
            )
        pl.semaphore_wait(barrier_sem, 2)

        def gemm_chunk(c):
            out_ref[pl.ds(c * ch, ch), :] = jnp.dot(
                x_ref[pl.ds(c * ch, ch), :], w_ref[...],
                preferred_element_type=jnp.float32,
            )

        gemm_chunk(my)
        for sub in range(SUB):
            rs_send("cw", 0, sub)
            rs_send("ccw", 0, sub)
        offs = [o for k in range(1, N_DEV // 2) for o in (-k, k)] + [N_DEV // 2]
        for o in offs:
            gemm_chunk((my + o) % N_DEV)

        for s in range(NSTEP):
            last = s == NSTEP - 1
            for sub in range(SUB):
                t = s * SUB + sub
                cw_recv_c = (my - s - 1) % N_DEV
                ccw_recv_c = (my + s + 1) % N_DEV
                rs_rdma("cw", t, cw_recv_c, sub).wait_recv()
                acc = out_ref[rows(cw_recv_c, sub), lo] + cw_ref[t % NSLOT]
                if last:
                    acc = silu(acc)
                    seed_cw_ref[sub] = acc.astype(jnp.bfloat16)
                out_ref[rows(cw_recv_c, sub), lo] = acc
                if last:
                    ag_send("cw", 0, sub)
                else:
                    rs_send("cw", s + 1, sub)
                rs_rdma("ccw", t, ccw_recv_c, sub).wait_recv()
                acc = out_ref[rows(ccw_recv_c, sub), hi] + ccw_ref[t % NSLOT]
                if last:
                    acc = silu(acc)
                    seed_ccw_ref[sub] = acc.astype(jnp.bfloat16)
                out_ref[rows(ccw_recv_c, sub), hi] = acc
                if last:
                    ag_send("ccw", 0, sub)
                else:
                    rs_send("ccw", s + 1, sub)
                if t + NSLOT <= NSUB - 1:
                    pl.semaphore_signal(
                        cw_credit_sems.at[t % NSLOT], inc=1,
                        device_id=(left,), device_id_type=pl.DeviceIdType.MESH,
                    )
                    pl.semaphore_signal(
                        ccw_credit_sems.at[t % NSLOT], inc=1,
                        device_id=(right,), device_id_type=pl.DeviceIdType.MESH,
                    )

        for s in range(NSTEP):
            for sub in range(SUB):
                t = s * SUB + sub
                cw_c = (my - s) % N_DEV
                ccw_c = (my + s) % N_DEV
                ag_rdma("cw", s, sub).wait_recv()
                out_ref[rows(cw_c, sub), lo] = (
                    ag_cw_ref[t % NSLOT].astype(jnp.float32)
                )
                if s + 1 < NSTEP:
                    ag_send("cw", s + 1, sub)
                ag_rdma("ccw", s, sub).wait_recv()
                out_ref[rows(ccw_c, sub), hi] = (
                    ag_ccw_ref[t % NSLOT].astype(jnp.float32)
                )
                if s + 1 < NSTEP:
                    ag_send("ccw", s + 1, sub)
                ag_rdma("cw", s, sub).wait_send()
                ag_rdma("ccw", s, sub).wait_send()
                if SUB <= t and t - SUB + NSLOT <= NSUB - 1:
                    pl.semaphore_signal(
                        ag_cw_credit_sems.at[(t - SUB) % NSLOT], inc=1,
                        device_id=(left,), device_id_type=pl.DeviceIdType.MESH,
                    )
                    pl.semaphore_signal(
                        ag_ccw_credit_sems.at[(t - SUB) % NSLOT], inc=1,
                        device_id=(right,), device_id_type=pl.DeviceIdType.MESH,
                    )

        for s in range(NSTEP):
            for sub in range(SUB):
                t = s * SUB + sub
                rs_rdma("cw", t, (my - s) % N_DEV, sub).wait_send()
                rs_rdma("ccw", t, (my + s) % N_DEV, sub).wait_send()

    return pl.pallas_call(
        body,
        out_shape=jax.ShapeDtypeStruct((m, n_out), jnp.float32),
        in_specs=[
            pl.BlockSpec(memory_space=pltpu.VMEM),
            pl.BlockSpec(memory_space=pltpu.VMEM),
        ],
        out_specs=pl.BlockSpec(memory_space=pltpu.VMEM),
        scratch_shapes=[
            pltpu.VMEM((NSLOT, sch, nh), jnp.float32),
            pltpu.VMEM((NSLOT, sch, nh), jnp.float32),
            pltpu.VMEM((NSLOT, sch, nh), jnp.bfloat16),
            pltpu.VMEM((NSLOT, sch, nh), jnp.bfloat16),
            pltpu.VMEM((SUB, sch, nh), jnp.bfloat16),
            pltpu.VMEM((SUB, sch, nh), jnp.bfloat16),
            pltpu.SemaphoreType.DMA((NSUB,)),
            pltpu.SemaphoreType.DMA((NSUB,)),
            pltpu.SemaphoreType.REGULAR((NSLOT,)),
            pltpu.SemaphoreType.DMA((NSUB,)),
            pltpu.SemaphoreType.DMA((NSUB,)),
            pltpu.SemaphoreType.REGULAR((NSLOT,)),
            pltpu.SemaphoreType.DMA((NSUB,)),
            pltpu.SemaphoreType.DMA((NSUB,)),
            pltpu.SemaphoreType.REGULAR((NSLOT,)),
            pltpu.SemaphoreType.DMA((NSUB,)),
            pltpu.SemaphoreType.DMA((NSUB,)),
            pltpu.SemaphoreType.REGULAR((NSLOT,)),
        ],
        compiler_params=pltpu.CompilerParams(
            collective_id=0,
            vmem_limit_bytes=60 * 1024 * 1024,
        ),
    )(x, w_mat)


# device time: 219565 ns/iter; 3.5458x vs baseline; 1.3847x over previous
import jax
import jax.numpy as jnp
from jax import lax
from jax.experimental import pallas as pl
from jax.experimental.pallas import tpu as pltpu

N_DEV = 16
SUB = 4
K_SLOTS = 4
NSLOT = K_SLOTS * SUB
NSTEP = N_DEV - 1
NSUB = NSTEP * SUB


def kernel(x, w_mat):
    m, k_shard = x.shape
    _, n_out = w_mat.shape
    ch = m // N_DEV
    sch = ch // SUB
    nh = n_out // 2

    def body(x_ref, w_ref, out_ref, cw_ref, ccw_ref,
             st_cw_ref, st_ccw_ref,
             ag_cw_ref, ag_ccw_ref, seed_cw_ref, seed_ccw_ref,
             cw_send_sems, cw_recv_sems, cw_credit_sems,
             ccw_send_sems, ccw_recv_sems, ccw_credit_sems,
             ag_cw_send_sems, ag_cw_recv_sems, ag_cw_credit_sems,
             ag_ccw_send_sems, ag_ccw_recv_sems, ag_ccw_credit_sems):
        my = lax.axis_index("i")
        left = (my - 1) % N_DEV
        right = (my + 1) % N_DEV

        lo = pl.ds(0, nh)
        hi = pl.ds(nh, nh)

        def rows(c, sub):
            return pl.ds(c * ch + sub * sch, sch)

        def silu(v):
            return v / (1.0 + jnp.exp(-jnp.clip(v, -60.0, 60.0)))

        def rs_rdma(direction, t):
            slot = t % NSLOT
            if direction == "cw":
                return pltpu.make_async_remote_copy(
                    src_ref=st_cw_ref.at[slot],
                    dst_ref=cw_ref.at[slot],
                    send_sem=cw_send_sems.at[t],
                    recv_sem=cw_recv_sems.at[t],
                    device_id=(right,),
                    device_id_type=pl.DeviceIdType.MESH,
                )
            return pltpu.make_async_remote_copy(
                src_ref=st_ccw_ref.at[slot],
                dst_ref=ccw_ref.at[slot],
                send_sem=ccw_send_sems.at[t],
                recv_sem=ccw_recv_sems.at[t],
                device_id=(left,),
                device_id_type=pl.DeviceIdType.MESH,
            )

        def rs_send(direction, s, sub, value):
            t = s * SUB + sub
            stage = st_cw_ref if direction == "cw" else st_ccw_ref
            stage[t % NSLOT] = value.astype(jnp.bfloat16)
            if t >= NSLOT:
                sems = cw_credit_sems if direction == "cw" else ccw_credit_sems
                pl.semaphore_wait(sems.at[t % NSLOT], 1)
            rs_rdma(direction, t).start()

        def ag_rdma(direction, s, sub):
            t = s * SUB + sub
            if direction == "cw":
                src = (seed_cw_ref.at[sub] if s == 0
                       else ag_cw_ref.at[(t - SUB) % NSLOT])
                return pltpu.make_async_remote_copy(
                    src_ref=src,
                    dst_ref=ag_cw_ref.at[t % NSLOT],
                    send_sem=ag_cw_send_sems.at[t],
                    recv_sem=ag_cw_recv_sems.at[t],
                    device_id=(right,),
                    device_id_type=pl.DeviceIdType.MESH,
                )
            src = (seed_ccw_ref.at[sub] if s == 0
                   else ag_ccw_ref.at[(t - SUB) % NSLOT])
            return pltpu.make_async_remote_copy(
                src_ref=src,
                dst_ref=ag_ccw_ref.at[t % NSLOT],
                send_sem=ag_ccw_send_sems.at[t],
                recv_sem=ag_ccw_recv_sems.at[t],
                device_id=(left,),
                device_id_type=pl.DeviceIdType.MESH,
            )

        def ag_send(direction, s, sub):
            t = s * SUB + sub
            if t >= NSLOT:
                sems = (ag_cw_credit_sems if direction == "cw"
                        else ag_ccw_credit_sems)
                pl.semaphore_wait(sems.at[t % NSLOT], 1)
            ag_rdma(direction, s, sub).start()

        barrier_sem = pltpu.get_barrier_semaphore()
        for nbr in (left, right):
            pl.semaphore_signal(
                barrier_sem, inc=1,
                device_id=(nbr,), device_id_type=pl.DeviceIdType.MESH,
            )
        pl.semaphore_wait(barrier_sem, 2)

        def gemm_chunk(c):
            out_ref[pl.ds(c * ch, ch), :] = jnp.dot(
                x_ref[pl.ds(c * ch, ch), :], w_ref[...],
                preferred_element_type=jnp.float32,
            )

        gemm_chunk(my)
        for sub in range(SUB):
            rs_send("cw", 0, sub, out_ref[rows(my, sub), lo])
            rs_send("ccw", 0, sub, out_ref[rows(my, sub), hi])
        offs = [o for k in range(1, N_DEV // 2) for o in (-k, k)] + [N_DEV // 2]
        for o in offs:
            gemm_chunk((my + o) % N_DEV)

        for s in range(NSTEP):
            last = s == NSTEP - 1
            for sub in range(SUB):
                t = s * SUB + sub
                cw_recv_c = (my - s - 1) % N_DEV
                ccw_recv_c = (my + s + 1) % N_DEV
                if t >= NSLOT - SUB:
                    rs_rdma("cw", t - (NSLOT - SUB)).wait_send()
                    rs_rdma("ccw", t - (NSLOT - SUB)).wait_send()
                rs_rdma("cw", t).wait_recv()
                acc = (out_ref[rows(cw_recv_c, sub), lo]
                       + cw_ref[t % NSLOT].astype(jnp.float32))
                if last:
                    acc = silu(acc)
                    out_ref[rows(cw_recv_c, sub), lo] = acc
                    seed_cw_ref[sub] = acc.astype(jnp.bfloat16)
                    ag_send("cw", 0, sub)
                else:
                    rs_send("cw", s + 1, sub, acc)
                rs_rdma("ccw", t).wait_recv()
                acc = (out_ref[rows(ccw_recv_c, sub), hi]
                       + ccw_ref[t % NSLOT].astype(jnp.float32))
                if last:
                    acc = silu(acc)
                    out_ref[rows(ccw_recv_c, sub), hi] = acc
                    seed_ccw_ref[sub] = acc.astype(jnp.bfloat16)
                    ag_send("ccw", 0, sub)
                else:
                    rs_send("ccw", s + 1, sub, acc)
                if t + NSLOT <= NSUB - 1:
                    pl.semaphore_signal(
                        cw_credit_sems.at[t % NSLOT], inc=1,
                        device_id=(left,), device_id_type=pl.DeviceIdType.MESH,
                    )
                    pl.semaphore_signal(
                        ccw_credit_sems.at[t % NSLOT], inc=1,
                        device_id=(right,), device_id_type=pl.DeviceIdType.MESH,
                    )

        for s in range(NSTEP):
            for sub in range(SUB):
                t = s * SUB + sub
                cw_c = (my - s) % N_DEV
                ccw_c = (my + s) % N_DEV
                ag_rdma("cw", s, sub).wait_recv()
                out_ref[rows(cw_c, sub), lo] = (
                    ag_cw_ref[t % NSLOT].astype(jnp.float32)
                )
                if s + 1 < NSTEP:
                    ag_send("cw", s + 1, sub)
                ag_rdma("ccw", s, sub).wait_recv()
                out_ref[rows(ccw_c, sub), hi] = (
                    ag_ccw_ref[t % NSLOT].astype(jnp.float32)
                )
                if s + 1 < NSTEP:
                    ag_send("ccw", s + 1, sub)
                ag_rdma("cw", s, sub).wait_send()
                ag_rdma("ccw", s, sub).wait_send()
                if SUB <= t and t - SUB + NSLOT <= NSUB - 1:
                    pl.semaphore_signal(
                        ag_cw_credit_sems.at[(t - SUB) % NSLOT], inc=1,
                        device_id=(left,), device_id_type=pl.DeviceIdType.MESH,
                    )
                    pl.semaphore_signal(
                        ag_ccw_credit_sems.at[(t - SUB) % NSLOT], inc=1,
                        device_id=(right,), device_id_type=pl.DeviceIdType.MESH,
                    )

        for t in range(NSUB - (NSLOT - SUB), NSUB):
            rs_rdma("cw", t).wait_send()
            rs_rdma("ccw", t).wait_send()

    return pl.pallas_call(
        body,
        out_shape=jax.ShapeDtypeStruct((m, n_out), jnp.float32),
        in_specs=[
            pl.BlockSpec(memory_space=pltpu.VMEM),
            pl.BlockSpec(memory_space=pltpu.VMEM),
        ],
        out_specs=pl.BlockSpec(memory_space=pltpu.VMEM),
        scratch_shapes=[
            pltpu.VMEM((NSLOT, sch, nh), jnp.bfloat16),
            pltpu.VMEM((NSLOT, sch, nh), jnp.bfloat16),
            pltpu.VMEM((NSLOT, sch, nh), jnp.bfloat16),
            pltpu.VMEM((NSLOT, sch, nh), jnp.bfloat16),
            pltpu.VMEM((NSLOT, sch, nh), jnp.bfloat16),
            pltpu.VMEM((NSLOT, sch, nh), jnp.bfloat16),
            pltpu.VMEM((SUB, sch, nh), jnp.bfloat16),
            pltpu.VMEM((SUB, sch, nh), jnp.bfloat16),
            pltpu.SemaphoreType.DMA((NSUB,)),
            pltpu.SemaphoreType.DMA((NSUB,)),
            pltpu.SemaphoreType.REGULAR((NSLOT,)),
            pltpu.SemaphoreType.DMA((NSUB,)),
            pltpu.SemaphoreType.DMA((NSUB,)),
            pltpu.SemaphoreType.REGULAR((NSLOT,)),
            pltpu.SemaphoreType.DMA((NSUB,)),
            pltpu.SemaphoreType.DMA((NSUB,)),
            pltpu.SemaphoreType.REGULAR((NSLOT,)),
            pltpu.SemaphoreType.DMA((NSUB,)),
            pltpu.SemaphoreType.DMA((NSUB,)),
            pltpu.SemaphoreType.REGULAR((NSLOT,)),
        ],
        compiler_params=pltpu.CompilerParams(
            collective_id=0,
            vmem_limit_bytes=60 * 1024 * 1024,
        ),
    )(x, w_mat)


# device time: 216812 ns/iter; 3.5909x vs baseline; 1.0127x over previous
import jax
import jax.numpy as jnp
from jax import lax
from jax.experimental import pallas as pl
from jax.experimental.pallas import tpu as pltpu

N_DEV = 16
SUB = 2
K_SLOTS = 4
NSLOT = K_SLOTS * SUB
NSTEP = N_DEV - 1
NSUB = NSTEP * SUB


def kernel(x, w_mat):
    m, k_shard = x.shape
    _, n_out = w_mat.shape
    ch = m // N_DEV
    sch = ch // SUB
    nh = n_out // 2

    def body(x_ref, w_ref, out_ref, cw_ref, ccw_ref,
             st_cw_ref, st_ccw_ref,
             ag_cw_ref, ag_ccw_ref, seed_cw_ref, seed_ccw_ref,
             cw_send_sems, cw_recv_sems, cw_credit_sems,
             ccw_send_sems, ccw_recv_sems, ccw_credit_sems,
             ag_cw_send_sems, ag_cw_recv_sems, ag_cw_credit_sems,
             ag_ccw_send_sems, ag_ccw_recv_sems, ag_ccw_credit_sems):
        my = lax.axis_index("i")
        left = (my - 1) % N_DEV
        right = (my + 1) % N_DEV

        lo = pl.ds(0, nh)
        hi = pl.ds(nh, nh)

        def rows(c, sub):
            return pl.ds(c * ch + sub * sch, sch)

        def silu(v):
            return v / (1.0 + jnp.exp(-jnp.clip(v, -60.0, 60.0)))

        def rs_rdma(direction, t):
            slot = t % NSLOT
            if direction == "cw":
                return pltpu.make_async_remote_copy(
                    src_ref=st_cw_ref.at[slot],
                    dst_ref=cw_ref.at[slot],
                    send_sem=cw_send_sems.at[t],
                    recv_sem=cw_recv_sems.at[t],
                    device_id=(right,),
                    device_id_type=pl.DeviceIdType.MESH,
                )
            return pltpu.make_async_remote_copy(
                src_ref=st_ccw_ref.at[slot],
                dst_ref=ccw_ref.at[slot],
                send_sem=ccw_send_sems.at[t],
                recv_sem=ccw_recv_sems.at[t],
                device_id=(left,),
                device_id_type=pl.DeviceIdType.MESH,
            )

        def rs_send(direction, s, sub, value):
            t = s * SUB + sub
            stage = st_cw_ref if direction == "cw" else st_ccw_ref
            stage[t % NSLOT] = value.astype(jnp.bfloat16)
            if t >= NSLOT:
                sems = cw_credit_sems if direction == "cw" else ccw_credit_sems
                pl.semaphore_wait(sems.at[t % NSLOT], 1)
            rs_rdma(direction, t).start()

        def ag_rdma(direction, s, sub):
            t = s * SUB + sub
            if direction == "cw":
                src = (seed_cw_ref.at[sub] if s == 0
                       else ag_cw_ref.at[(t - SUB) % NSLOT])
                return pltpu.make_async_remote_copy(
                    src_ref=src,
                    dst_ref=ag_cw_ref.at[t % NSLOT],
                    send_sem=ag_cw_send_sems.at[t],
                    recv_sem=ag_cw_recv_sems.at[t],
                    device_id=(right,),
                    device_id_type=pl.DeviceIdType.MESH,
                )
            src = (seed_ccw_ref.at[sub] if s == 0
                   else ag_ccw_ref.at[(t - SUB) % NSLOT])
            return pltpu.make_async_remote_copy(
                src_ref=src,
                dst_ref=ag_ccw_ref.at[t % NSLOT],
                send_sem=ag_ccw_send_sems.at[t],
                recv_sem=ag_ccw_recv_sems.at[t],
                device_id=(left,),
                device_id_type=pl.DeviceIdType.MESH,
            )

        def ag_send(direction, s, sub):
            t = s * SUB + sub
            if t >= NSLOT:
                sems = (ag_cw_credit_sems if direction == "cw"
                        else ag_ccw_credit_sems)
                pl.semaphore_wait(sems.at[t % NSLOT], 1)
            ag_rdma(direction, s, sub).start()

        barrier_sem = pltpu.get_barrier_semaphore()
        for nbr in (left, right):
            pl.semaphore_signal(
                barrier_sem, inc=1,
                device_id=(nbr,), device_id_type=pl.DeviceIdType.MESH,
            )
        pl.semaphore_wait(barrier_sem, 2)

        def gemm_chunk(c):
            out_ref[pl.ds(c * ch, ch), :] = jnp.dot(
                x_ref[pl.ds(c * ch, ch), :], w_ref[...],
                preferred_element_type=jnp.float32,
            )

        gemm_chunk(my)
        for sub in range(SUB):
            rs_send("cw", 0, sub, out_ref[rows(my, sub), lo])
            rs_send("ccw", 0, sub, out_ref[rows(my, sub), hi])
        offs = [o for k in range(1, N_DEV // 2) for o in (-k, k)] + [N_DEV // 2]
        for o in offs:
            gemm_chunk((my + o) % N_DEV)

        for s in range(NSTEP):
            last = s == NSTEP - 1
            for sub in range(SUB):
                t = s * SUB + sub
                cw_recv_c = (my - s - 1) % N_DEV
                ccw_recv_c = (my + s + 1) % N_DEV
                if t >= NSLOT - SUB:
                    rs_rdma("cw", t - (NSLOT - SUB)).wait_send()
                    rs_rdma("ccw", t - (NSLOT - SUB)).wait_send()
                rs_rdma("cw", t).wait_recv()
                acc = (out_ref[rows(cw_recv_c, sub), lo]
                       + cw_ref[t % NSLOT].astype(jnp.float32))
                if last:
                    acc = silu(acc)
                    out_ref[rows(cw_recv_c, sub), lo] = acc
                    seed_cw_ref[sub] = acc.astype(jnp.bfloat16)
                    ag_send("cw", 0, sub)
                else:
                    rs_send("cw", s + 1, sub, acc)
                rs_rdma("ccw", t).wait_recv()
                acc = (out_ref[rows(ccw_recv_c, sub), hi]
                       + ccw_ref[t % NSLOT].astype(jnp.float32))
                if last:
                    acc = silu(acc)
                    out_ref[rows(ccw_recv_c, sub), hi] = acc
                    seed_ccw_ref[sub] = acc.astype(jnp.bfloat16)
                    ag_send("ccw", 0, sub)
                else:
                    rs_send("ccw", s + 1, sub, acc)
                if t + NSLOT <= NSUB - 1:
                    pl.semaphore_signal(
                        cw_credit_sems.at[t % NSLOT], inc=1,
                        device_id=(left,), device_id_type=pl.DeviceIdType.MESH,
                    )
                    pl.semaphore_signal(
                        ccw_credit_sems.at[t % NSLOT], inc=1,
                        device_id=(right,), device_id_type=pl.DeviceIdType.MESH,
                    )

        for s in range(NSTEP):
            for sub in range(SUB):
                t = s * SUB + sub
                cw_c = (my - s) % N_DEV
                ccw_c = (my + s) % N_DEV
                ag_rdma("cw", s, sub).wait_recv()
                out_ref[rows(cw_c, sub), lo] = (
                    ag_cw_ref[t % NSLOT].astype(jnp.float32)
                )
                if s + 1 < NSTEP:
                    ag_send("cw", s + 1, sub)
                ag_rdma("ccw", s, sub).wait_recv()
                out_ref[rows(ccw_c, sub), hi] = (
                    ag_ccw_ref[t % NSLOT].astype(jnp.float32)
                )
                if s + 1 < NSTEP:
                    ag_send("ccw", s + 1, sub)
                ag_rdma("cw", s, sub).wait_send()
                ag_rdma("ccw", s, sub).wait_send()
                if SUB <= t and t - SUB + NSLOT <= NSUB - 1:
                    pl.semaphore_signal(
                        ag_cw_credit_sems.at[(t - SUB) % NSLOT], inc=1,
                        device_id=(left,), device_id_type=pl.DeviceIdType.MESH,
                    )
                    pl.semaphore_signal(
                        ag_ccw_credit_sems.at[(t - SUB) % NSLOT], inc=1,
                        device_id=(right,), device_id_type=pl.DeviceIdType.MESH,
                    )

        for t in range(NSUB - (NSLOT - SUB), NSUB):
            rs_rdma("cw", t).wait_send()
            rs_rdma("ccw", t).wait_send()

    return pl.pallas_call(
        body,
        out_shape=jax.ShapeDtypeStruct((m, n_out), jnp.float32),
        in_specs=[
            pl.BlockSpec(memory_space=pltpu.VMEM),
            pl.BlockSpec(memory_space=pltpu.VMEM),
        ],
        out_specs=pl.BlockSpec(memory_space=pltpu.VMEM),
        scratch_shapes=[
            pltpu.VMEM((NSLOT, sch, nh), jnp.bfloat16),
            pltpu.VMEM((NSLOT, sch, nh), jnp.bfloat16),
            pltpu.VMEM((NSLOT, sch, nh), jnp.bfloat16),
            pltpu.VMEM((NSLOT, sch, nh), jnp.bfloat16),
            pltpu.VMEM((NSLOT, sch, nh), jnp.bfloat16),
            pltpu.VMEM((NSLOT, sch, nh), jnp.bfloat16),
            pltpu.VMEM((SUB, sch, nh), jnp.bfloat16),
            pltpu.VMEM((SUB, sch, nh), jnp.bfloat16),
            pltpu.SemaphoreType.DMA((NSUB,)),
            pltpu.SemaphoreType.DMA((NSUB,)),
            pltpu.SemaphoreType.REGULAR((NSLOT,)),
            pltpu.SemaphoreType.DMA((NSUB,)),
            pltpu.SemaphoreType.DMA((NSUB,)),
            pltpu.SemaphoreType.REGULAR((NSLOT,)),
            pltpu.SemaphoreType.DMA((NSUB,)),
            pltpu.SemaphoreType.DMA((NSUB,)),
            pltpu.SemaphoreType.REGULAR((NSLOT,)),
            pltpu.SemaphoreType.DMA((NSUB,)),
            pltpu.SemaphoreType.DMA((NSUB,)),
            pltpu.SemaphoreType.REGULAR((NSLOT,)),
        ],
        compiler_params=pltpu.CompilerParams(
            collective_id=0,
            vmem_limit_bytes=60 * 1024 * 1024,
        ),
    )(x, w_mat)
